# baseline (speedup 1.0000x reference)
import jax
import jax.numpy as jnp
from jax.experimental import pallas as pl
from jax.experimental.pallas import tpu as pltpu


def _body(pos_ref, kv_ref, cache_ref, out_ref, copy_sem, tok_sem):
    pos = pos_ref[0]
    cp = pltpu.make_async_copy(cache_ref, out_ref, copy_sem)
    cp.start()
    cp.wait()
    tok = pltpu.make_async_copy(kv_ref, out_ref.at[:, pl.ds(pos, 1), :], tok_sem)
    tok.start()
    tok.wait()


def kernel(kv, start_pos, kv_cache):
    bsz = kv.shape[0]
    win = kv_cache.shape[1]
    pos = jnp.reshape(jnp.asarray(start_pos, jnp.int32) % win, (1,))
    cache = kv_cache[:bsz]
    out = pl.pallas_call(
        _body,
        out_shape=jax.ShapeDtypeStruct(cache.shape, cache.dtype),
        in_specs=[
            pl.BlockSpec(memory_space=pltpu.SMEM),
            pl.BlockSpec(memory_space=pltpu.HBM),
            pl.BlockSpec(memory_space=pltpu.HBM),
        ],
        out_specs=pl.BlockSpec(memory_space=pltpu.HBM),
        scratch_shapes=[pltpu.SemaphoreType.DMA, pltpu.SemaphoreType.DMA],
    )(pos, kv, cache)
    return out


# speedup vs baseline: 48.1079x; 48.1079x over previous
import jax
import jax.numpy as jnp
from jax.experimental import pallas as pl
from jax.experimental.pallas import tpu as pltpu

_B_BLK = 8
_W_BLK = 1024


def _body(pos_ref, kv_ref, cache_ref, out_ref):
    j = pl.program_id(1)
    local = pos_ref[0] - j * _W_BLK
    hit = (local >= 0) & (local < _W_BLK)

    @pl.when(hit)
    def _():
        ids = jax.lax.broadcasted_iota(jnp.int32, cache_ref.shape, 1)
        out_ref[...] = jnp.where(ids == local, kv_ref[...], cache_ref[...])

    @pl.when(jnp.logical_not(hit))
    def _():
        out_ref[...] = cache_ref[...]


def kernel(kv, start_pos, kv_cache):
    bsz, _, head = kv.shape
    win = kv_cache.shape[1]
    pos = jnp.reshape(jnp.asarray(start_pos, jnp.int32) % win, (1,))
    cache = kv_cache[:bsz]
    out = pl.pallas_call(
        _body,
        grid=(bsz // _B_BLK, win // _W_BLK),
        out_shape=jax.ShapeDtypeStruct(cache.shape, cache.dtype),
        in_specs=[
            pl.BlockSpec(memory_space=pltpu.SMEM),
            pl.BlockSpec((_B_BLK, 1, head), lambda i, j: (i, 0, 0)),
            pl.BlockSpec((_B_BLK, _W_BLK, head), lambda i, j: (i, j, 0)),
        ],
        out_specs=pl.BlockSpec((_B_BLK, _W_BLK, head), lambda i, j: (i, j, 0)),
    )(pos, kv, cache)
    return out


# speedup vs baseline: 48.9565x; 1.0176x over previous
import jax
import jax.numpy as jnp
from jax.experimental import pallas as pl
from jax.experimental.pallas import tpu as pltpu

_B_BLK = 8
_W_BLK = 2048


def _body(pos_ref, kv_ref, cache_ref, out_ref):
    j = pl.program_id(1)
    local = pos_ref[0] - j * _W_BLK
    hit = (local >= 0) & (local < _W_BLK)

    @pl.when(hit)
    def _():
        ids = jax.lax.broadcasted_iota(jnp.int32, cache_ref.shape, 1)
        out_ref[...] = jnp.where(ids == local, kv_ref[...], cache_ref[...])

    @pl.when(jnp.logical_not(hit))
    def _():
        out_ref[...] = cache_ref[...]


def kernel(kv, start_pos, kv_cache):
    bsz, _, head = kv.shape
    win = kv_cache.shape[1]
    pos = jnp.reshape(jnp.asarray(start_pos, jnp.int32) % win, (1,))
    cache = kv_cache[:bsz]
    out = pl.pallas_call(
        _body,
        grid=(bsz // _B_BLK, win // _W_BLK),
        out_shape=jax.ShapeDtypeStruct(cache.shape, cache.dtype),
        in_specs=[
            pl.BlockSpec(memory_space=pltpu.SMEM),
            pl.BlockSpec((_B_BLK, 1, head), lambda i, j: (i, 0, 0)),
            pl.BlockSpec((_B_BLK, _W_BLK, head), lambda i, j: (i, j, 0)),
        ],
        out_specs=pl.BlockSpec((_B_BLK, _W_BLK, head), lambda i, j: (i, j, 0)),
    )(pos, kv, cache)
    return out
